# initial kernel scaffold (unmeasured)
import jax
import jax.numpy as jnp
from jax import lax
from jax.experimental import pallas as pl
from jax.experimental.pallas import tpu as pltpu

N_DEV = 4
B, SQ, SKV_L, HQ, DH = 2, 512, 512, 32, 64
H_L = HQ // N_DEV
SKV = SKV_L * N_DEV
D_MODEL = 768
BLK = 64

_DIT = getattr(pl, "DeviceIdType", None) or pltpu.DeviceIdType
_sem_signal = getattr(pl, "semaphore_signal", None) or pltpu.semaphore_signal
_sem_wait = getattr(pl, "semaphore_wait", None) or pltpu.semaphore_wait
_CP = getattr(pltpu, "CompilerParams", None) or pltpu.TPUCompilerParams


def kernel(x, Wq, K_ext, V_ext, Wo):
    def body(x_ref, wq_ref, k_ref, v_ref, wo_ref, out_ref,
             q_buf, kg, vg, ctx_buf, comm,
             kr_sems, vr_sems, ks_sems, vs_sems, ar_send, ar_recv):
        me = lax.axis_index("i")
        right = lax.rem(me + 1, N_DEV)

        barrier_sem = pltpu.get_barrier_semaphore()
        for d in range(1, N_DEV):
            peer = lax.rem(me + d, N_DEV)
            _sem_signal(barrier_sem, inc=1, device_id=(peer,),
                        device_id_type=_DIT.MESH)
        _sem_wait(barrier_sem, N_DEV - 1)

        sends = []
        for d in range(1, N_DEV):
            j = lax.rem(me + d, N_DEV)
            for src_ref, dst_g, ssem, rsem in (
                (k_ref, kg, ks_sems, kr_sems),
                (v_ref, vg, vs_sems, vr_sems),
            ):
                rdma = pltpu.make_async_remote_copy(
                    src_ref=src_ref.at[:, :, pl.ds(j * H_L, H_L), :],
                    dst_ref=dst_g.at[:, pl.ds(me * SKV_L, SKV_L)],
                    send_sem=ssem.at[j],
                    recv_sem=rsem.at[me],
                    device_id=(j,),
                    device_id_type=_DIT.MESH,
                )
                rdma.start()
                sends.append(rdma)

        kg[:, pl.ds(me * SKV_L, SKV_L)] = k_ref[:, :, pl.ds(me * H_L, H_L), :]
        vg[:, pl.ds(me * SKV_L, SKV_L)] = v_ref[:, :, pl.ds(me * H_L, H_L), :]

        for b in range(B):
            q_buf[b] = jnp.dot(x_ref[b], wq_ref[...],
                               preferred_element_type=jnp.float32)

        qb = lax.broadcasted_iota(jnp.int32, (SQ, SKV), 0) // BLK
        kb = lax.broadcasted_iota(jnp.int32, (SQ, SKV), 1) // BLK
        mask = (qb == kb) | (kb == 0) | (lax.rem(qb + kb, 3) == 0)
        bias = jnp.where(mask, 0.0, -1e9).astype(jnp.float32)

        for d in range(1, N_DEV):
            j = lax.rem(me + d, N_DEV)
            for dst_g, ssem, rsem in ((kg, ks_sems, kr_sems),
                                      (vg, vs_sems, vr_sems)):
                recv = pltpu.make_async_remote_copy(
                    src_ref=dst_g.at[:, pl.ds(j * SKV_L, SKV_L)],
                    dst_ref=dst_g.at[:, pl.ds(j * SKV_L, SKV_L)],
                    send_sem=ssem.at[j],
                    recv_sem=rsem.at[j],
                    device_id=(j,),
                    device_id_type=_DIT.MESH,
                )
                recv.wait_recv()

        for b in range(B):
            for h in range(H_L):
                q = q_buf[b, :, h * DH:(h + 1) * DH]
                k = kg[b, :, h, :]
                s = lax.dot_general(
                    q, k, (((1,), (1,)), ((), ())),
                    preferred_element_type=jnp.float32,
                ) * 0.125 + bias
                m = jnp.max(s, axis=1, keepdims=True)
                w = jnp.exp(s - m)
                w = w / jnp.sum(w, axis=1, keepdims=True)
                ctx_buf[b, :, h * DH:(h + 1) * DH] = jnp.dot(
                    w, vg[b, :, h, :], preferred_element_type=jnp.float32)

        for b in range(B):
            p = jnp.dot(ctx_buf[b], wo_ref[...],
                        preferred_element_type=jnp.float32)
            comm[0, b] = p
            out_ref[b] = p

        for rdma in sends:
            rdma.wait_send()

        for h in range(N_DEV - 1):
            rdma = pltpu.make_async_remote_copy(
                src_ref=comm.at[h],
                dst_ref=comm.at[h + 1],
                send_sem=ar_send.at[h],
                recv_sem=ar_recv.at[h],
                device_id=(right,),
                device_id_type=_DIT.MESH,
            )
            rdma.start()
            rdma.wait()
            out_ref[...] += comm[h + 1]

    return pl.pallas_call(
        body,
        out_shape=jax.ShapeDtypeStruct((B, SQ, D_MODEL), jnp.float32),
        in_specs=[pl.BlockSpec(memory_space=pltpu.VMEM)] * 5,
        out_specs=pl.BlockSpec(memory_space=pltpu.VMEM),
        scratch_shapes=[
            pltpu.VMEM((B, SQ, H_L * DH), jnp.float32),
            pltpu.VMEM((B, SKV, H_L, DH), jnp.float32),
            pltpu.VMEM((B, SKV, H_L, DH), jnp.float32),
            pltpu.VMEM((B, SQ, H_L * DH), jnp.float32),
            pltpu.VMEM((N_DEV, B, SQ, D_MODEL), jnp.float32),
            pltpu.SemaphoreType.DMA((N_DEV,)),
            pltpu.SemaphoreType.DMA((N_DEV,)),
            pltpu.SemaphoreType.DMA((N_DEV,)),
            pltpu.SemaphoreType.DMA((N_DEV,)),
            pltpu.SemaphoreType.DMA((N_DEV - 1,)),
            pltpu.SemaphoreType.DMA((N_DEV - 1,)),
        ],
        compiler_params=_CP(collective_id=0),
    )(x, Wq, K_ext, V_ext, Wo)


# baseline (device time: 144974 ns/iter reference)
import jax
import jax.numpy as jnp
from jax import lax
from jax.experimental import pallas as pl
from jax.experimental.pallas import tpu as pltpu

N_DEV = 4
B, SQ, SKV_L, HQ, DH = 2, 512, 512, 32, 64
H_L = HQ // N_DEV
HD_L = H_L * DH
SKV = SKV_L * N_DEV
D_MODEL = 768
BLK = 64
C = SQ // N_DEV
QT = 128
N_QT = SQ // QT

_DIT = getattr(pl, "DeviceIdType", None) or pltpu.DeviceIdType
_sem_signal = getattr(pl, "semaphore_signal", None) or pltpu.semaphore_signal
_sem_wait = getattr(pl, "semaphore_wait", None) or pltpu.semaphore_wait
_CP = getattr(pltpu, "CompilerParams", None) or pltpu.TPUCompilerParams


def kernel(x, Wq, K_ext, V_ext, Wo):
    def body(x_ref, wq_ref, k_ref, v_ref, wo_ref, out_ref,
             q_buf, kg, vg, ctx_buf, bias_ref, pbuf, comm,
             kr_sems, vr_sems, ks_sems, vs_sems,
             rs_send, rs_recv, ag_send, ag_recv):
        me = lax.axis_index("i")
        right = lax.rem(me + 1, N_DEV)

        barrier_sem = pltpu.get_barrier_semaphore()
        for d in range(1, N_DEV):
            peer = lax.rem(me + d, N_DEV)
            _sem_signal(barrier_sem, inc=1, device_id=(peer,),
                        device_id_type=_DIT.MESH)
        _sem_wait(barrier_sem, N_DEV - 1)

        sends = []
        for d in range(1, N_DEV):
            j = lax.rem(me + d, N_DEV)
            for src_ref, dst_g, ssem, rsem in (
                (k_ref, kg, ks_sems, kr_sems),
                (v_ref, vg, vs_sems, vr_sems),
            ):
                rdma = pltpu.make_async_remote_copy(
                    src_ref=src_ref.at[:, :, pl.ds(j * HD_L, HD_L)],
                    dst_ref=dst_g.at[:, pl.ds(me * SKV_L, SKV_L), :],
                    send_sem=ssem.at[j],
                    recv_sem=rsem.at[me],
                    device_id=(j,),
                    device_id_type=_DIT.MESH,
                )
                rdma.start()
                sends.append(rdma)

        kg[:, pl.ds(me * SKV_L, SKV_L), :] = k_ref[:, :, pl.ds(me * HD_L, HD_L)]
        vg[:, pl.ds(me * SKV_L, SKV_L), :] = v_ref[:, :, pl.ds(me * HD_L, HD_L)]

        for b in range(B):
            q_buf[b] = jnp.dot(x_ref[b], wq_ref[...],
                               preferred_element_type=jnp.float32)

        qb = lax.broadcasted_iota(jnp.int32, (SQ, SKV), 0) // BLK
        kb = lax.broadcasted_iota(jnp.int32, (SQ, SKV), 1) // BLK
        mask = (qb == kb) | (kb == 0) | (lax.rem(qb + kb, 3) == 0)
        bias_ref[...] = jnp.where(mask, 0.0, -1e9).astype(jnp.float32)

        for d in range(1, N_DEV):
            j = lax.rem(me + d, N_DEV)
            for dst_g, ssem, rsem in ((kg, ks_sems, kr_sems),
                                      (vg, vs_sems, vr_sems)):
                recv = pltpu.make_async_remote_copy(
                    src_ref=dst_g.at[:, pl.ds(j * SKV_L, SKV_L), :],
                    dst_ref=dst_g.at[:, pl.ds(j * SKV_L, SKV_L), :],
                    send_sem=ssem.at[j],
                    recv_sem=rsem.at[j],
                    device_id=(j,),
                    device_id_type=_DIT.MESH,
                )
                recv.wait_recv()

        for b in range(B):
            for h in range(H_L):
                k = kg[b, :, h * DH:(h + 1) * DH]
                v = vg[b, :, h * DH:(h + 1) * DH]
                for t in range(N_QT):
                    q = q_buf[b, t * QT:(t + 1) * QT,
                              h * DH:(h + 1) * DH].astype(jnp.bfloat16)
                    s = lax.dot_general(
                        q, k, (((1,), (1,)), ((), ())),
                        preferred_element_type=jnp.float32,
                    ) * 0.125 + bias_ref[t * QT:(t + 1) * QT, :]
                    m = jnp.max(s, axis=1, keepdims=True)
                    w = jnp.exp(s - m)
                    w = w / jnp.sum(w, axis=1, keepdims=True)
                    ctx_buf[b, t * QT:(t + 1) * QT,
                            h * DH:(h + 1) * DH] = jnp.dot(
                        w.astype(jnp.bfloat16), v,
                        preferred_element_type=jnp.float32)

        for b in range(B):
            pbuf[b] = jnp.dot(ctx_buf[b], wo_ref[...],
                              preferred_element_type=jnp.float32
                              ).astype(jnp.bfloat16)

        for rdma in sends:
            rdma.wait_send()

        comm[0] = pbuf[:, pl.ds(me * C, C), :]
        for s in range(N_DEV - 1):
            rdma = pltpu.make_async_remote_copy(
                src_ref=comm.at[s],
                dst_ref=comm.at[s + 1],
                send_sem=rs_send.at[s],
                recv_sem=rs_recv.at[s],
                device_id=(right,),
                device_id_type=_DIT.MESH,
            )
            rdma.start()
            rdma.wait()
            c = lax.rem(me - s - 1 + N_DEV, N_DEV)
            comm[s + 1] += pbuf[:, pl.ds(c * C, C), :]

        own = lax.rem(me + 1, N_DEV)
        pbuf[:, pl.ds(own * C, C), :] = comm[N_DEV - 1]

        for g in range(N_DEV - 1):
            c = lax.rem(me + 1 - g + N_DEV, N_DEV)
            rdma = pltpu.make_async_remote_copy(
                src_ref=pbuf.at[:, pl.ds(c * C, C), :],
                dst_ref=pbuf.at[:, pl.ds(c * C, C), :],
                send_sem=ag_send.at[g],
                recv_sem=ag_recv.at[g],
                device_id=(right,),
                device_id_type=_DIT.MESH,
            )
            rdma.start()
            rdma.wait()

        out_ref[...] = pbuf[...].astype(jnp.float32)

    kv_shape = (B, SKV_L, HQ * DH)
    return pl.pallas_call(
        body,
        out_shape=jax.ShapeDtypeStruct((B, SQ, D_MODEL), jnp.float32),
        in_specs=[pl.BlockSpec(memory_space=pltpu.VMEM)] * 5,
        out_specs=pl.BlockSpec(memory_space=pltpu.VMEM),
        scratch_shapes=[
            pltpu.VMEM((B, SQ, HD_L), jnp.float32),
            pltpu.VMEM((B, SKV, HD_L), jnp.bfloat16),
            pltpu.VMEM((B, SKV, HD_L), jnp.bfloat16),
            pltpu.VMEM((B, SQ, HD_L), jnp.float32),
            pltpu.VMEM((SQ, SKV), jnp.float32),
            pltpu.VMEM((B, SQ, D_MODEL), jnp.bfloat16),
            pltpu.VMEM((N_DEV, B, C, D_MODEL), jnp.bfloat16),
            pltpu.SemaphoreType.DMA((N_DEV,)),
            pltpu.SemaphoreType.DMA((N_DEV,)),
            pltpu.SemaphoreType.DMA((N_DEV,)),
            pltpu.SemaphoreType.DMA((N_DEV,)),
            pltpu.SemaphoreType.DMA((N_DEV - 1,)),
            pltpu.SemaphoreType.DMA((N_DEV - 1,)),
            pltpu.SemaphoreType.DMA((N_DEV - 1,)),
            pltpu.SemaphoreType.DMA((N_DEV - 1,)),
        ],
        compiler_params=_CP(collective_id=0),
    )(x, Wq,
      K_ext.reshape(kv_shape).astype(jnp.bfloat16),
      V_ext.reshape(kv_shape).astype(jnp.bfloat16),
      Wo)


# device time: 125657 ns/iter; 1.1537x vs baseline; 1.1537x over previous
import jax
import jax.numpy as jnp
from jax import lax
from jax.experimental import pallas as pl
from jax.experimental.pallas import tpu as pltpu

N_DEV = 4
B, SQ, SKV_L, HQ, DH = 2, 512, 512, 32, 64
H_L = HQ // N_DEV
HD_L = H_L * DH
SKV = SKV_L * N_DEV
D_MODEL = 768
BLK = 64
C = SQ // N_DEV

_DIT = getattr(pl, "DeviceIdType", None) or pltpu.DeviceIdType
_sem_signal = getattr(pl, "semaphore_signal", None) or pltpu.semaphore_signal
_sem_wait = getattr(pl, "semaphore_wait", None) or pltpu.semaphore_wait
_CP = getattr(pltpu, "CompilerParams", None) or pltpu.TPUCompilerParams

BF = jnp.bfloat16


def kernel(x, Wq, K_ext, V_ext, Wo):
    def body(x_ref, wq_ref, k_ref, v_ref, wo_ref, out_ref,
             q_buf, kg, vg, ctx_buf, acc_buf, lbuf, bias4, psend, ar1_buf,
             pbuf, kr_sems, vr_sems, ks_sems, vs_sems,
             ar1_s, ar1_r, ar2_s, ar2_r):
        me = lax.axis_index("i")

        barrier_sem = pltpu.get_barrier_semaphore()
        for d in range(1, N_DEV):
            peer = lax.rem(me + d, N_DEV)
            _sem_signal(barrier_sem, inc=1, device_id=(peer,),
                        device_id_type=_DIT.MESH)
        _sem_wait(barrier_sem, N_DEV - 1)

        sends = []
        for d in range(1, N_DEV):
            j = lax.rem(me + d, N_DEV)
            for src_ref, dst_g, ssem, rsem in (
                (k_ref, kg, ks_sems, kr_sems),
                (v_ref, vg, vs_sems, vr_sems),
            ):
                rdma = pltpu.make_async_remote_copy(
                    src_ref=src_ref.at[:, :, pl.ds(j * HD_L, HD_L)],
                    dst_ref=dst_g.at[:, pl.ds(me * SKV_L, SKV_L), :],
                    send_sem=ssem.at[j],
                    recv_sem=rsem.at[me],
                    device_id=(j,),
                    device_id_type=_DIT.MESH,
                )
                rdma.start()
                sends.append(rdma)

        kg[:, pl.ds(me * SKV_L, SKV_L), :] = k_ref[:, :, pl.ds(me * HD_L, HD_L)]
        vg[:, pl.ds(me * SKV_L, SKV_L), :] = v_ref[:, :, pl.ds(me * HD_L, HD_L)]

        for b in range(B):
            q_buf[b] = jnp.dot(x_ref[b], wq_ref[...],
                               preferred_element_type=jnp.float32).astype(BF)

        for c in range(N_DEV):
            qb = lax.broadcasted_iota(jnp.int32, (SQ, SKV_L), 0) // BLK
            kb = lax.broadcasted_iota(jnp.int32, (SQ, SKV_L), 1) // BLK + 8 * c
            mask = (qb == kb) | (kb == 0) | (lax.rem(qb + kb, 3) == 0)
            bias4[c] = jnp.where(mask, 0.0, -1e9).astype(jnp.float32)

        for idx, d in enumerate((0, 1, 3, 2)):
            j = lax.rem(me + d, N_DEV)
            if d != 0:
                for dst_g, ssem, rsem in ((kg, ks_sems, kr_sems),
                                          (vg, vs_sems, vr_sems)):
                    recv = pltpu.make_async_remote_copy(
                        src_ref=dst_g.at[:, pl.ds(j * SKV_L, SKV_L), :],
                        dst_ref=dst_g.at[:, pl.ds(j * SKV_L, SKV_L), :],
                        send_sem=ssem.at[j],
                        recv_sem=rsem.at[j],
                        device_id=(j,),
                        device_id_type=_DIT.MESH,
                    )
                    recv.wait_recv()
            for b in range(B):
                for h in range(H_L):
                    k = kg[b, pl.ds(j * SKV_L, SKV_L),
                           h * DH:(h + 1) * DH]
                    v = vg[b, pl.ds(j * SKV_L, SKV_L),
                           h * DH:(h + 1) * DH]
                    for t in range(SQ // C):
                        ts = slice(t * C, (t + 1) * C)
                        q = q_buf[b, ts, h * DH:(h + 1) * DH]
                        s = lax.dot_general(
                            q, k, (((1,), (1,)), ((), ())),
                            preferred_element_type=jnp.float32,
                        ) * 0.125 + bias4[j, ts, :]
                        w = jnp.exp(s)
                        lw = jnp.sum(w, axis=1, keepdims=True)
                        aw = jnp.dot(w.astype(BF), v,
                                     preferred_element_type=jnp.float32)
                        if idx == 0:
                            acc_buf[b, ts, h * DH:(h + 1) * DH] = aw
                            lbuf[b, h, ts] = lw
                        else:
                            acc_buf[b, ts, h * DH:(h + 1) * DH] += aw
                            lbuf[b, h, ts] += lw

        for b in range(B):
            for h in range(H_L):
                ctx_buf[b, :, h * DH:(h + 1) * DH] = (
                    acc_buf[b, :, h * DH:(h + 1) * DH] / lbuf[b, h]
                ).astype(BF)

        for d in range(1, N_DEV):
            t = lax.rem(me + d, N_DEV)
            for b in range(B):
                psend[d - 1, b] = jnp.dot(
                    ctx_buf[b, pl.ds(t * C, C), :], wo_ref[...],
                    preferred_element_type=jnp.float32).astype(BF)
            rdma = pltpu.make_async_remote_copy(
                src_ref=psend.at[d - 1],
                dst_ref=ar1_buf.at[d - 1],
                send_sem=ar1_s.at[d - 1],
                recv_sem=ar1_r.at[d - 1],
                device_id=(t,),
                device_id_type=_DIT.MESH,
            )
            rdma.start()
            sends.append(rdma)
        for b in range(B):
            psend[N_DEV - 1, b] = jnp.dot(
                ctx_buf[b, pl.ds(me * C, C), :], wo_ref[...],
                preferred_element_type=jnp.float32).astype(BF)

        for d in range(1, N_DEV):
            j = lax.rem(me - d + N_DEV, N_DEV)
            recv = pltpu.make_async_remote_copy(
                src_ref=ar1_buf.at[d - 1], dst_ref=ar1_buf.at[d - 1],
                send_sem=ar1_s.at[d - 1], recv_sem=ar1_r.at[d - 1],
                device_id=(j,), device_id_type=_DIT.MESH,
            )
            recv.wait_recv()
        red = (psend[N_DEV - 1].astype(jnp.float32)
               + ar1_buf[0].astype(jnp.float32)
               + ar1_buf[1].astype(jnp.float32)
               + ar1_buf[2].astype(jnp.float32)).astype(BF)
        pbuf[:, pl.ds(me * C, C), :] = red

        for d in range(1, N_DEV):
            t = lax.rem(me + d, N_DEV)
            rdma = pltpu.make_async_remote_copy(
                src_ref=pbuf.at[:, pl.ds(me * C, C), :],
                dst_ref=pbuf.at[:, pl.ds(me * C, C), :],
                send_sem=ar2_s.at[d - 1],
                recv_sem=ar2_r.at[d - 1],
                device_id=(t,),
                device_id_type=_DIT.MESH,
            )
            rdma.start()
            sends.append(rdma)
        for d in range(1, N_DEV):
            j = lax.rem(me - d + N_DEV, N_DEV)
            recv = pltpu.make_async_remote_copy(
                src_ref=pbuf.at[:, pl.ds(j * C, C), :],
                dst_ref=pbuf.at[:, pl.ds(j * C, C), :],
                send_sem=ar2_s.at[d - 1], recv_sem=ar2_r.at[d - 1],
                device_id=(j,), device_id_type=_DIT.MESH,
            )
            recv.wait_recv()

        out_ref[...] = pbuf[...].astype(jnp.float32)

        for rdma in sends:
            rdma.wait_send()

    kv_shape = (B, SKV_L, HQ * DH)
    return pl.pallas_call(
        body,
        out_shape=jax.ShapeDtypeStruct((B, SQ, D_MODEL), jnp.float32),
        in_specs=[pl.BlockSpec(memory_space=pltpu.VMEM)] * 5,
        out_specs=pl.BlockSpec(memory_space=pltpu.VMEM),
        scratch_shapes=[
            pltpu.VMEM((B, SQ, HD_L), BF),
            pltpu.VMEM((B, SKV, HD_L), BF),
            pltpu.VMEM((B, SKV, HD_L), BF),
            pltpu.VMEM((B, SQ, HD_L), BF),
            pltpu.VMEM((B, SQ, HD_L), jnp.float32),
            pltpu.VMEM((B, H_L, SQ, 1), jnp.float32),
            pltpu.VMEM((N_DEV, SQ, SKV_L), jnp.float32),
            pltpu.VMEM((N_DEV, B, C, D_MODEL), BF),
            pltpu.VMEM((N_DEV - 1, B, C, D_MODEL), BF),
            pltpu.VMEM((B, SQ, D_MODEL), BF),
            pltpu.SemaphoreType.DMA((N_DEV,)),
            pltpu.SemaphoreType.DMA((N_DEV,)),
            pltpu.SemaphoreType.DMA((N_DEV,)),
            pltpu.SemaphoreType.DMA((N_DEV,)),
            pltpu.SemaphoreType.DMA((N_DEV - 1,)),
            pltpu.SemaphoreType.DMA((N_DEV - 1,)),
            pltpu.SemaphoreType.DMA((N_DEV - 1,)),
            pltpu.SemaphoreType.DMA((N_DEV - 1,)),
        ],
        compiler_params=_CP(collective_id=0),
    )(x.astype(BF), Wq.astype(BF),
      K_ext.reshape(kv_shape).astype(BF),
      V_ext.reshape(kv_shape).astype(BF),
      Wo.astype(BF))
